# baseline (speedup 1.0000x reference)
import functools

import jax
import jax.numpy as jnp
from jax import lax
from jax.experimental import pallas as pl
from jax.experimental.pallas import tpu as pltpu
from jax.experimental.pallas import tpu_sc as plsc

N_IN = 128
EMB_DIM = 512
TOPO_W = EMB_DIM - N_IN
WAVE = 8


def kernel(inputs, grid_positions, embedding, topographical_embedding, x_learn, y_learn):
    B, GRID, _ = inputs.shape
    info = plsc.get_sparse_core_info()
    NC, NS = info.num_cores, info.num_subcores
    NW = NC * NS
    ROWS = GRID // NW

    mesh = plsc.VectorSubcoreMesh(core_axis_name="c", subcore_axis_name="s")

    @functools.partial(
        pl.kernel,
        mesh=mesh,
        out_type=jax.ShapeDtypeStruct((B * GRID, 1, EMB_DIM), jnp.float32),
        scratch_types=[
            pltpu.VMEM((ROWS, 1, TOPO_W), jnp.float32),
            pltpu.SemaphoreType.DMA,
            pltpu.SemaphoreType.DMA,
        ],
    )
    def sc_embed(in_hbm, topo_hbm, out_hbm, tbuf, sem_t, sem_i):
        wid = lax.axis_index("s") * NC + lax.axis_index("c")
        g0 = wid * ROWS

        pltpu.sync_copy(
            topo_hbm.at[pl.ds(g0, ROWS), pl.ds(0, TOPO_W)],
            tbuf.at[:, 0, :],
        )

        def topo_out(b):
            return pltpu.make_async_copy(
                tbuf,
                out_hbm.at[pl.ds(b * GRID + g0, ROWS), :, pl.ds(N_IN, TOPO_W)],
                sem_t,
            )

        def in_out(b):
            return pltpu.make_async_copy(
                in_hbm.at[b, pl.ds(g0, ROWS), :],
                out_hbm.at[pl.ds(b * GRID + g0, ROWS), 0, pl.ds(0, N_IN)],
                sem_i,
            )

        for w0 in range(0, B, WAVE):
            for b in range(w0, w0 + WAVE):
                topo_out(b).start()
                in_out(b).start()
            for b in range(w0, w0 + WAVE):
                topo_out(b).wait()
                in_out(b).wait()

    return sc_embed(inputs, topographical_embedding)


# speedup vs baseline: 9.3140x; 9.3140x over previous
import functools

import jax
import jax.numpy as jnp
from jax import lax
from jax.experimental import pallas as pl
from jax.experimental.pallas import tpu as pltpu
from jax.experimental.pallas import tpu_sc as plsc

N_IN = 128
EMB_DIM = 512
TOPO_W = EMB_DIM - N_IN
NBUF = 7
PF = 3


def kernel(inputs, grid_positions, embedding, topographical_embedding, x_learn, y_learn):
    B, GRID, _ = inputs.shape
    info = plsc.get_sparse_core_info()
    NC, NS = info.num_cores, info.num_subcores
    NW = NC * NS
    ROWS = GRID // NW

    mesh = plsc.VectorSubcoreMesh(core_axis_name="c", subcore_axis_name="s")

    @functools.partial(
        pl.kernel,
        mesh=mesh,
        out_type=jax.ShapeDtypeStruct((B * GRID, 1, EMB_DIM), jnp.float32),
        scratch_types=(
            [pltpu.VMEM((ROWS, 1, EMB_DIM), jnp.float32) for _ in range(NBUF)]
            + [pltpu.SemaphoreType.DMA for _ in range(2 * NBUF)]
        ),
    )
    def sc_embed(in_hbm, topo_hbm, out_hbm, *scratch):
        bufs = scratch[:NBUF]
        in_sems = scratch[NBUF : 2 * NBUF]
        out_sems = scratch[2 * NBUF :]
        wid = lax.axis_index("s") * NC + lax.axis_index("c")
        g0 = wid * ROWS

        def topo_copy(k):
            return pltpu.make_async_copy(
                topo_hbm.at[pl.ds(g0, ROWS), pl.ds(0, TOPO_W)],
                bufs[k].at[:, 0, pl.ds(N_IN, TOPO_W)],
                out_sems[k],
            )

        for k in range(NBUF):
            topo_copy(k).start()
        for k in range(NBUF):
            topo_copy(k).wait()

        def in_copy(b):
            return pltpu.make_async_copy(
                in_hbm.at[b, pl.ds(g0, ROWS), :],
                bufs[b % NBUF].at[:, 0, pl.ds(0, N_IN)],
                in_sems[b % NBUF],
            )

        def out_copy(b):
            return pltpu.make_async_copy(
                bufs[b % NBUF],
                out_hbm.at[pl.ds(b * GRID + g0, ROWS)],
                out_sems[b % NBUF],
            )

        out_waited = [False] * B
        for b in range(PF):
            in_copy(b).start()
        for b in range(B):
            nb = b + PF
            if nb < B:
                prev = nb - NBUF
                if prev >= 0:
                    out_copy(prev).wait()
                    out_waited[prev] = True
                in_copy(nb).start()
            in_copy(b).wait()
            out_copy(b).start()
        for b in range(B):
            if not out_waited[b]:
                out_copy(b).wait()

    return sc_embed(inputs, topographical_embedding)


# speedup vs baseline: 9.5117x; 1.0212x over previous
import functools

import jax
import jax.numpy as jnp
from jax import lax
from jax.experimental import pallas as pl
from jax.experimental.pallas import tpu as pltpu
from jax.experimental.pallas import tpu_sc as plsc

N_IN = 128
EMB_DIM = 512
TOPO_W = EMB_DIM - N_IN
NBUF = 6
PF = 4


def kernel(inputs, grid_positions, embedding, topographical_embedding, x_learn, y_learn):
    B, GRID, _ = inputs.shape
    info = plsc.get_sparse_core_info()
    NC, NS = info.num_cores, info.num_subcores
    NW = NC * NS
    ROWS = GRID // NW

    mesh = plsc.VectorSubcoreMesh(core_axis_name="c", subcore_axis_name="s")

    @functools.partial(
        pl.kernel,
        mesh=mesh,
        out_type=jax.ShapeDtypeStruct((B * GRID, 1, EMB_DIM), jnp.float32),
        scratch_types=(
            [pltpu.VMEM((ROWS, 1, EMB_DIM), jnp.float32) for _ in range(NBUF)]
            + [pltpu.SemaphoreType.DMA for _ in range(2 * NBUF)]
        ),
    )
    def sc_embed(in_hbm, topo_hbm, out_hbm, *scratch):
        bufs = scratch[:NBUF]
        in_sems = scratch[NBUF : 2 * NBUF]
        out_sems = scratch[2 * NBUF :]
        wid = lax.axis_index("s") * NC + lax.axis_index("c")
        g0 = wid * ROWS

        def topo_copy(k):
            return pltpu.make_async_copy(
                topo_hbm.at[pl.ds(g0, ROWS), pl.ds(0, TOPO_W)],
                bufs[k].at[:, 0, pl.ds(N_IN, TOPO_W)],
                out_sems[k],
            )

        for k in range(NBUF):
            topo_copy(k).start()
        for k in range(NBUF):
            topo_copy(k).wait()

        def in_copy(b):
            return pltpu.make_async_copy(
                in_hbm.at[b, pl.ds(g0, ROWS), :],
                bufs[b % NBUF].at[:, 0, pl.ds(0, N_IN)],
                in_sems[b % NBUF],
            )

        def out_copy(b):
            return pltpu.make_async_copy(
                bufs[b % NBUF],
                out_hbm.at[pl.ds(b * GRID + g0, ROWS)],
                out_sems[b % NBUF],
            )

        out_waited = [False] * B
        for b in range(PF):
            in_copy(b).start()
        for b in range(B):
            nb = b + PF
            if nb < B:
                prev = nb - NBUF
                if prev >= 0:
                    out_copy(prev).wait()
                    out_waited[prev] = True
                in_copy(nb).start()
            in_copy(b).wait()
            out_copy(b).start()
        for b in range(B):
            if not out_waited[b]:
                out_copy(b).wait()

    return sc_embed(inputs, topographical_embedding)
